# baseline (speedup 1.0000x reference)
import functools

import jax
import jax.numpy as jnp
from jax import lax
from jax.experimental import pallas as pl
from jax.experimental.pallas import tpu as pltpu
from jax.experimental.pallas import tpu_sc as plsc

_B = 16384
_D = 64
_K = 20
_NC = 2
_NS = 16
_NW = _NC * _NS
_BPW = _B // _NW
_CB = 32
_BB = 8
_NBLK = _BPW // _BB
_NGR = _CB * _K // 128
_WNGR = _BPW * _K // 128

_DYN_DNUMS = lax.GatherDimensionNumbers(
    offset_dims=(), collapsed_slice_dims=(0,), start_index_map=(0,))


def _bcast_lane(vec, idx_splat):
    return lax.gather(vec, idx_splat[:, None], _DYN_DNUMS, slice_sizes=(1,),
                      mode=lax.GatherScatterMode.PROMISE_IN_BOUNDS)


def _sc_dots_body(c_hbm, o_hbm, ng_hbm, v_hbm, u_hbm, sp_hbm, sn_hbm,
                  c_v, o_v, ngr_v, cpi_v, opi_v, npi_v, vc_b, uo_b, un_b,
                  pt, pt_sp, sp_res, sn_res, sem):
    wid = lax.axis_index("s") * _NC + lax.axis_index("c")
    base = wid * _BPW
    pltpu.sync_copy(c_hbm.at[pl.ds(base, _BPW)], c_v.at[pl.ds(0, _BPW)])
    pltpu.sync_copy(o_hbm.at[pl.ds(base, _BPW)], o_v.at[pl.ds(0, _BPW)])
    pltpu.sync_copy(ng_hbm.at[pl.ds(wid * _WNGR, _WNGR), :], ngr_v)

    lane = lax.iota(jnp.int32, 16)
    cols = [jnp.full((16,), cc, jnp.int32) for cc in range(16)]
    colbase = [lane + 16 * j for j in range(4)]

    def lane_sums(ptref, rows):
        acc = plsc.load_gather(ptref, [rows, cols[0]])
        for cc in range(1, 16):
            acc = acc + plsc.load_gather(ptref, [rows, cols[cc]])
        return acc

    def block(blk, carry):
        ch = blk // 4
        q4 = blk % 4

        @pl.when(q4 == 0)
        def _dma():
            for t in range(2):
                cpi_v[pl.ds(16 * t, 16)] = lax.shift_right_logical(
                    c_v[pl.ds(ch * _CB + 16 * t, 16)], 1)
                opi_v[pl.ds(16 * t, 16)] = lax.shift_right_logical(
                    o_v[pl.ds(ch * _CB + 16 * t, 16)], 1)
            for r in range(_NGR):
                for t in range(8):
                    npi_v[r, pl.ds(16 * t, 16)] = lax.shift_right_logical(
                        ngr_v[ch * _NGR + r, pl.ds(16 * t, 16)], 1)
            cp_vc = pltpu.async_copy(v_hbm.at[cpi_v], vc_b, sem)
            cp_uo = pltpu.async_copy(u_hbm.at[opi_v], uo_b, sem)
            cps = []
            for r in range(_NGR):
                cps.append(pltpu.async_copy(
                    u_hbm.at[npi_v.at[r]],
                    un_b.at[pl.ds(r * 128, 128), :], sem))
            cp_vc.wait()
            cp_uo.wait()
            for cp in cps:
                cp.wait()

        rowb = q4 * _BB
        unb = q4 * _BB * _K
        hcv = (c_v[pl.ds(blk * _BB, 16)] & 1) * 64
        hov = (o_v[pl.ds(blk * _BB, 16)] & 1) * 64

        vcreg = []
        for b in range(_BB):
            hb = _bcast_lane(hcv, cols[b])
            rows_b = jnp.full((16,), rowb + b, jnp.int32)
            vcreg.append([plsc.load_gather(vc_b, [rows_b, colbase[j] + hb])
                          for j in range(4)])

        for b in range(_BB):
            hb = _bcast_lane(hov, cols[b])
            rows_b = jnp.full((16,), rowb + b, jnp.int32)
            part = vcreg[b][0] * plsc.load_gather(
                uo_b, [rows_b, colbase[0] + hb])
            for j in range(1, 4):
                part = part + vcreg[b][j] * plsc.load_gather(
                    uo_b, [rows_b, colbase[j] + hb])
            pt_sp[(blk % 2) * _BB + b, pl.ds(0, 16)] = part

        @pl.when(blk % 2 == 1)
        def _spflush():
            sp_res[pl.ds((blk // 2) * 16, 16)] = lane_sums(pt_sp, lane)

        for g in range(10):
            pos = blk * _BB * _K + g * 16
            ngl = ngr_v[pos // 128, pl.ds(pos % 128, 16)]
            hnv = (ngl & 1) * 64
            slot = g % 4
            for i in range(16):
                q = g * 16 + i
                lb = q // _K
                hb = _bcast_lane(hnv, cols[i])
                rows_q = jnp.full((16,), unb + q, jnp.int32)
                part = vcreg[lb][0] * plsc.load_gather(
                    un_b, [rows_q, colbase[0] + hb])
                for j in range(1, 4):
                    part = part + vcreg[lb][j] * plsc.load_gather(
                        un_b, [rows_q, colbase[j] + hb])
                pt[slot * 16 + i, pl.ds(0, 16)] = part
            rv = lane_sums(pt, slot * 16 + lane)
            sn_res[pl.ds(blk * _BB * _K + g * 16, 16)] = rv
        return carry

    lax.fori_loop(0, _NBLK, block, 0)

    pltpu.sync_copy(sp_res, sp_hbm.at[pl.ds(base, _BPW)])
    pltpu.sync_copy(sn_res, sn_hbm.at[pl.ds(wid * _BPW * _K, _BPW * _K)])


_sc_dots = functools.partial(
    pl.kernel,
    out_type=(jax.ShapeDtypeStruct((_B,), jnp.float32),
              jax.ShapeDtypeStruct((_B * _K,), jnp.float32)),
    mesh=plsc.VectorSubcoreMesh(core_axis_name="c", subcore_axis_name="s"),
    compiler_params=pltpu.CompilerParams(
        needs_layout_passes=False, use_tc_tiling_on_sc=True),
    scratch_types=[
        pltpu.VMEM((_BPW + 16,), jnp.int32),
        pltpu.VMEM((_BPW + 16,), jnp.int32),
        pltpu.VMEM((_WNGR, 128), jnp.int32),
        pltpu.VMEM((_CB,), jnp.int32),
        pltpu.VMEM((_CB,), jnp.int32),
        pltpu.VMEM((_NGR, 128), jnp.int32),
        pltpu.VMEM((_CB, 128), jnp.float32),
        pltpu.VMEM((_CB, 128), jnp.float32),
        pltpu.VMEM((_CB * _K, 128), jnp.float32),
        pltpu.VMEM((64, 17), jnp.float32),
        pltpu.VMEM((16, 17), jnp.float32),
        pltpu.VMEM((_BPW,), jnp.float32),
        pltpu.VMEM((_BPW * _K,), jnp.float32),
        pltpu.SemaphoreType.DMA,
    ],
)(_sc_dots_body)


def _logsig(x):
    return jnp.minimum(x, 0.0) - jnp.log1p(jnp.exp(-jnp.abs(x)))


def _loss_body(sp_ref, sn_ref, out_ref):
    lp = _logsig(sp_ref[...])
    ln = _logsig(-sn_ref[...])
    out_ref[...] = jnp.reshape(-(jnp.sum(lp) + jnp.sum(ln)) / _B, (1, 1))


def kernel(c, o, ng, V, U):
    ng2 = ng.reshape(_B * _K // 128, 128)
    vp = V.reshape(V.shape[0] // 2, 2 * _D)
    up = U.reshape(U.shape[0] // 2, 2 * _D)
    sp, sn = _sc_dots(c, o, ng2, vp, up)
    loss = pl.pallas_call(
        _loss_body,
        out_shape=jax.ShapeDtypeStruct((1, 1), jnp.float32),
    )(sp.reshape(128, 128), sn.reshape(_B * _K // 128, 128))
    return loss[0, 0]


# speedup vs baseline: 1.1601x; 1.1601x over previous
import functools

import jax
import jax.numpy as jnp
from jax import lax
from jax.experimental import pallas as pl
from jax.experimental.pallas import tpu as pltpu
from jax.experimental.pallas import tpu_sc as plsc

_B = 16384
_D = 64
_K = 20
_NC = 2
_NS = 16
_NW = _NC * _NS
_BPW = _B // _NW
_CB = 16
_NCHUNK = _BPW // _CB
_IDXW = 64
_NGROWS = _BPW * _K // _IDXW
_ROWS_PER_CHUNK = _CB * _K // _IDXW


def _sc_dots_body(c_hbm, o_hbm, ng_hbm, v_hbm, u_hbm, sp_hbm, sn_hbm,
                  c_v, o_v, ng_v, vc_b, uo_b, un_b, pt, pt_sp,
                  sp_res, sn_res, sem0, sem1):
    wid = lax.axis_index("s") * _NC + lax.axis_index("c")
    base = wid * _BPW
    pltpu.sync_copy(c_hbm.at[pl.ds(base, _BPW)], c_v)
    pltpu.sync_copy(o_hbm.at[pl.ds(base, _BPW)], o_v)
    pltpu.sync_copy(ng_hbm.at[pl.ds(wid * _NGROWS, _NGROWS), :], ng_v)

    lane = lax.iota(jnp.int32, 16)
    cols = [jnp.full((16,), cc, jnp.int32) for cc in range(16)]
    sems = (sem0, sem1)

    def lane_sums(ptref, rows):
        acc = plsc.load_gather(ptref, [rows, cols[0]])
        for cc in range(1, 16):
            acc = acc + plsc.load_gather(ptref, [rows, cols[cc]])
        return acc

    def issue(ch, slot):
        sem = sems[slot]
        pltpu.async_copy(v_hbm.at[c_v.at[pl.ds(ch * _CB, _CB)]],
                         vc_b.at[pl.ds(slot * _CB, _CB), :], sem)
        pltpu.async_copy(u_hbm.at[o_v.at[pl.ds(ch * _CB, _CB)]],
                         uo_b.at[pl.ds(slot * _CB, _CB), :], sem)
        for r in range(_ROWS_PER_CHUNK):
            pltpu.async_copy(
                u_hbm.at[ng_v.at[ch * _ROWS_PER_CHUNK + r]],
                un_b.at[pl.ds(slot * _CB * _K + r * _IDXW, _IDXW), :], sem)

    def drain(slot):
        pltpu.make_async_copy(v_hbm.at[pl.ds(0, _CB), :],
                              vc_b.at[pl.ds(slot * _CB, _CB), :],
                              sems[slot]).wait()
        pltpu.make_async_copy(u_hbm.at[pl.ds(0, _CB), :],
                              uo_b.at[pl.ds(slot * _CB, _CB), :],
                              sems[slot]).wait()
        pltpu.make_async_copy(
            u_hbm.at[pl.ds(0, _CB * _K), :],
            un_b.at[pl.ds(slot * _CB * _K, _CB * _K), :], sems[slot]).wait()

    issue(0, 0)

    def chunk(ch, carry):
        for s in (0, 1):
            @pl.when(jnp.logical_and(ch % 2 == s, ch + 1 < _NCHUNK))
            def _issue_next(s=s):
                issue(ch + 1, 1 - s)

            @pl.when(ch % 2 == s)
            def _drain_cur(s=s):
                drain(s)

        offb = (ch % 2) * _CB
        offu = (ch % 2) * _CB * _K
        for sb in range(4):
            vcreg = [[vc_b[offb + sb * 4 + b, pl.ds(16 * j, 16)]
                      for j in range(4)] for b in range(4)]
            for b in range(4):
                part = vcreg[b][0] * uo_b[offb + sb * 4 + b, pl.ds(0, 16)]
                for j in range(1, 4):
                    part = part + vcreg[b][j] * uo_b[offb + sb * 4 + b,
                                                     pl.ds(16 * j, 16)]
                pt_sp[sb * 4 + b, pl.ds(0, 16)] = part
            for g in range(5):
                slot = g % 4
                for i in range(16):
                    q = g * 16 + i
                    p = sb * 80 + q
                    lb = q // _K
                    part = vcreg[lb][0] * un_b[offu + p, pl.ds(0, 16)]
                    for j in range(1, 4):
                        part = part + vcreg[lb][j] * un_b[offu + p,
                                                          pl.ds(16 * j, 16)]
                    pt[slot * 16 + i, pl.ds(0, 16)] = part
                rv = lane_sums(pt, slot * 16 + lane)
                sn_res[pl.ds(ch * _CB * _K + sb * 80 + g * 16, 16)] = rv
        sp_res[pl.ds(ch * _CB, _CB)] = lane_sums(pt_sp, lane)
        return carry

    lax.fori_loop(0, _NCHUNK, chunk, 0)

    pltpu.sync_copy(sp_res, sp_hbm.at[pl.ds(base, _BPW)])
    pltpu.sync_copy(sn_res, sn_hbm.at[pl.ds(wid * _BPW * _K, _BPW * _K)])


_sc_dots = functools.partial(
    pl.kernel,
    out_type=(jax.ShapeDtypeStruct((_B,), jnp.float32),
              jax.ShapeDtypeStruct((_B * _K,), jnp.float32)),
    mesh=plsc.VectorSubcoreMesh(core_axis_name="c", subcore_axis_name="s"),
    compiler_params=pltpu.CompilerParams(
        needs_layout_passes=False, use_tc_tiling_on_sc=False),
    scratch_types=[
        pltpu.VMEM((_BPW,), jnp.int32),
        pltpu.VMEM((_BPW,), jnp.int32),
        pltpu.VMEM((_NGROWS, _IDXW), jnp.int32),
        pltpu.VMEM((2 * _CB, _D), jnp.float32),
        pltpu.VMEM((2 * _CB, _D), jnp.float32),
        pltpu.VMEM((2 * _CB * _K, _D), jnp.float32),
        pltpu.VMEM((64, 17), jnp.float32),
        pltpu.VMEM((16, 17), jnp.float32),
        pltpu.VMEM((_BPW,), jnp.float32),
        pltpu.VMEM((_BPW * _K,), jnp.float32),
        pltpu.SemaphoreType.DMA,
        pltpu.SemaphoreType.DMA,
    ],
)(_sc_dots_body)


def _logsig(x):
    return jnp.minimum(x, 0.0) - jnp.log1p(jnp.exp(-jnp.abs(x)))


def _loss_body(sp_ref, sn_ref, out_ref):
    lp = _logsig(sp_ref[...])
    ln = _logsig(-sn_ref[...])
    out_ref[...] = jnp.reshape(-(jnp.sum(lp) + jnp.sum(ln)) / _B, (1, 1))


def kernel(c, o, ng, V, U):
    ng2 = ng.reshape(_B * _K // _IDXW, _IDXW)
    sp, sn = _sc_dots(c, o, ng2, V, U)
    loss = pl.pallas_call(
        _loss_body,
        out_shape=jax.ShapeDtypeStruct((1, 1), jnp.float32),
    )(sp.reshape(128, 128), sn.reshape(_B * _K // 128, 128))
    return loss[0, 0]
